# baseline (speedup 1.0000x reference)
import functools

import jax
import jax.numpy as jnp
from jax.experimental import pallas as pl
from jax.experimental.pallas import tpu as pltpu

NH = 4
CODES = 1024
EMB = 1024
HD = EMB // NH
NTOK = 16384
BT = 2048
LOG2E = 1.4426950408889634


def _vq_kernel(scales_ref, z_ref, emb_ref, aux_ref, zq_ref, idx_ref,
               comb_ref, perp_ref, psum_ref, msk_ref, nidx_ref):
    t = pl.program_id(0)
    h = pl.program_id(1)

    @pl.when(jnp.logical_and(t == 0, h == 0))
    def _init():
        psum_ref[...] = jnp.zeros_like(psum_ref)

    z = z_ref[...]
    emb = emb_ref[0]

    zn = z / jnp.maximum(
        jnp.sqrt(jnp.sum(z * z, axis=-1, keepdims=True)), 1e-12)
    en = emb / jnp.maximum(
        jnp.sqrt(jnp.sum(emb * emb, axis=-1, keepdims=True)), 1e-12)

    raw = jax.lax.dot_general(
        zn, en, (((1,), (1,)), ((), ())),
        preferred_element_type=jnp.float32)

    m = jnp.max(raw, axis=-1, keepdims=True)
    msk_ref[...] = (raw >= m).astype(jnp.float32)

    aux = aux_ref[...]
    ic = jax.lax.dot_general(
        msk_ref[...], aux, (((1,), (0,)), ((), ())),
        preferred_element_type=jnp.float32)
    nidx_ref[...] = ic[:, 0:1]

    @pl.when(jnp.max(ic[:, 1]) > 1.5)
    def _fix_ties():
        niota = aux[:, 0].reshape(1, CODES)
        cand = jnp.where(raw >= m, niota, -3.0e38)
        widx = jnp.max(cand, axis=-1, keepdims=True)
        msk_ref[...] = (cand == widx).astype(jnp.float32)
        nidx_ref[...] = widx

    onehot = msk_ref[...]
    idx = (-nidx_ref[...][:, 0]).astype(jnp.int32)

    zq_ref[...] = jax.lax.dot_general(
        onehot, emb, (((1,), (0,)), ((), ())),
        preferred_element_type=jnp.float32)

    k = scales_ref[h] * LOG2E
    e = jnp.exp2(raw * k)
    s = jax.lax.dot_general(
        e, aux, (((1,), (0,)), ((), ())),
        preferred_element_type=jnp.float32)[:, 1:2]
    inv = 1.0 / s
    colsum = jax.lax.dot_general(
        inv, e, (((0,), (0,)), ((), ())),
        preferred_element_type=jnp.float32)
    psum_ref[h, :] = psum_ref[h, :] + colsum[0]

    idx_ref[0, 0, :] = idx

    @pl.when(h == 0)
    def _comb0():
        comb_ref[0, 0, :] = idx

    @pl.when(h > 0)
    def _combh():
        comb_ref[0, 0, :] = comb_ref[0, 0, :] * CODES + idx

    @pl.when(jnp.logical_and(t == pl.num_programs(0) - 1, h == NH - 1))
    def _finish():
        p = psum_ref[...] * (1.0 / NTOK)
        ent = jnp.sum(p * jnp.log(p + 1e-10), axis=-1, keepdims=True)
        perp_ref[0, 0] = jnp.mean(jnp.exp(-ent))


@functools.partial(jax.jit, static_argnames=())
def kernel(z_e, embeddings, logit_scales):
    nt = NTOK // BT
    grid = (nt, NH)
    aux = jnp.zeros((CODES, 128), dtype=jnp.float32)
    aux = aux.at[:, 0].set(-jnp.arange(CODES, dtype=jnp.float32))
    aux = aux.at[:, 1].set(1.0)
    zq, idx, comb, perp = pl.pallas_call(
        _vq_kernel,
        grid=grid,
        in_specs=[
            pl.BlockSpec(memory_space=pltpu.SMEM),
            pl.BlockSpec((BT, HD), lambda t, h: (t, h)),
            pl.BlockSpec((1, CODES, HD), lambda t, h: (h, 0, 0)),
            pl.BlockSpec((CODES, 128), lambda t, h: (0, 0)),
        ],
        out_specs=[
            pl.BlockSpec((BT, HD), lambda t, h: (t, h)),
            pl.BlockSpec((1, 1, BT), lambda t, h: (h, 0, t)),
            pl.BlockSpec((1, 1, BT), lambda t, h: (0, 0, t)),
            pl.BlockSpec((1, 1), lambda t, h: (0, 0),
                         memory_space=pltpu.SMEM),
        ],
        out_shape=[
            jax.ShapeDtypeStruct((NTOK, EMB), jnp.float32),
            jax.ShapeDtypeStruct((NH, 1, NTOK), jnp.int32),
            jax.ShapeDtypeStruct((1, 1, NTOK), jnp.int32),
            jax.ShapeDtypeStruct((1, 1), jnp.float32),
        ],
        scratch_shapes=[
            pltpu.VMEM((NH, CODES), jnp.float32),
            pltpu.VMEM((BT, CODES), jnp.float32),
            pltpu.VMEM((BT, 1), jnp.float32),
        ],
    )(logit_scales, z_e, embeddings, aux)

    temperature = jnp.asarray(1.0, dtype=jnp.float32)
    commitment_loss = jnp.asarray(0.0, dtype=jnp.float32)
    return (zq, comb[0, 0], perp[0, 0], temperature, commitment_loss)


# speedup vs baseline: 1.4826x; 1.4826x over previous
import functools

import jax
import jax.numpy as jnp
from jax.experimental import pallas as pl
from jax.experimental.pallas import tpu as pltpu

NH = 4
CODES = 1024
EMB = 1024
HD = EMB // NH
NTOK = 16384
BT = 2048
LOG2E = 1.4426950408889634


def _vq_kernel(scales_ref, z_ref, emb_ref, niota_ref, zq_ref, idx_ref,
               comb_ref, perp_ref, psum_ref):
    t = pl.program_id(0)
    h = pl.program_id(1)

    @pl.when(jnp.logical_and(t == 0, h == 0))
    def _init():
        psum_ref[...] = jnp.zeros_like(psum_ref)

    z = z_ref[...]
    emb = emb_ref[0]

    zn = z / jnp.maximum(
        jnp.sqrt(jnp.sum(z * z, axis=-1, keepdims=True)), 1e-12)
    en = emb / jnp.maximum(
        jnp.sqrt(jnp.sum(emb * emb, axis=-1, keepdims=True)), 1e-12)

    raw = jax.lax.dot_general(
        zn, en, (((1,), (1,)), ((), ())),
        preferred_element_type=jnp.float32)

    m = jnp.max(raw, axis=-1, keepdims=True)
    niota = niota_ref[...]

    cand = jnp.where(raw >= m, niota, -3.0e38)
    widx = jnp.max(cand, axis=-1, keepdims=True)
    idx = (-widx[:, 0]).astype(jnp.int32)

    onehot = (cand == widx).astype(jnp.float32)
    zq_ref[...] = jax.lax.dot_general(
        onehot, emb, (((1,), (0,)), ((), ())),
        preferred_element_type=jnp.float32)

    k = scales_ref[h] * LOG2E
    e = jnp.exp2(raw * k)
    inv = 1.0 / jnp.sum(e, axis=-1, keepdims=True)
    colsum = jax.lax.dot_general(
        inv, e, (((0,), (0,)), ((), ())),
        preferred_element_type=jnp.float32)
    psum_ref[h, :] = psum_ref[h, :] + colsum[0]

    idx_ref[0, 0, :] = idx

    @pl.when(h == 0)
    def _comb0():
        comb_ref[0, 0, :] = idx

    @pl.when(h > 0)
    def _combh():
        comb_ref[0, 0, :] = comb_ref[0, 0, :] * CODES + idx

    @pl.when(jnp.logical_and(t == pl.num_programs(0) - 1, h == NH - 1))
    def _finish():
        p = psum_ref[...] * (1.0 / NTOK)
        ent = jnp.sum(p * jnp.log(p + 1e-10), axis=-1, keepdims=True)
        perp_ref[0, 0] = jnp.mean(jnp.exp(-ent))


@functools.partial(jax.jit, static_argnames=())
def kernel(z_e, embeddings, logit_scales):
    nt = NTOK // BT
    grid = (nt, NH)
    niota = -jnp.arange(CODES, dtype=jnp.float32).reshape(1, CODES)
    zq, idx, comb, perp = pl.pallas_call(
        _vq_kernel,
        grid=grid,
        in_specs=[
            pl.BlockSpec(memory_space=pltpu.SMEM),
            pl.BlockSpec((BT, HD), lambda t, h: (t, h)),
            pl.BlockSpec((1, CODES, HD), lambda t, h: (h, 0, 0)),
            pl.BlockSpec((1, CODES), lambda t, h: (0, 0)),
        ],
        out_specs=[
            pl.BlockSpec((BT, HD), lambda t, h: (t, h)),
            pl.BlockSpec((1, 1, BT), lambda t, h: (h, 0, t)),
            pl.BlockSpec((1, 1, BT), lambda t, h: (0, 0, t)),
            pl.BlockSpec((1, 1), lambda t, h: (0, 0),
                         memory_space=pltpu.SMEM),
        ],
        out_shape=[
            jax.ShapeDtypeStruct((NTOK, EMB), jnp.float32),
            jax.ShapeDtypeStruct((NH, 1, NTOK), jnp.int32),
            jax.ShapeDtypeStruct((1, 1, NTOK), jnp.int32),
            jax.ShapeDtypeStruct((1, 1), jnp.float32),
        ],
        scratch_shapes=[pltpu.VMEM((NH, CODES), jnp.float32)],
    )(logit_scales, z_e, embeddings, niota)

    temperature = jnp.asarray(1.0, dtype=jnp.float32)
    commitment_loss = jnp.asarray(0.0, dtype=jnp.float32)
    return (zq, comb[0, 0], perp[0, 0], temperature, commitment_loss)


# speedup vs baseline: 1.5736x; 1.0614x over previous
import functools

import jax
import jax.numpy as jnp
from jax.experimental import pallas as pl
from jax.experimental.pallas import tpu as pltpu

NH = 4
CODES = 1024
EMB = 1024
HD = EMB // NH
NTOK = 16384
BT = 4096
LOG2E = 1.4426950408889634


def _vq_kernel(scales_ref, z_ref, emb_ref, niota_ref, zq_ref, idx_ref,
               comb_ref, perp_ref, psum_ref):
    t = pl.program_id(0)
    h = pl.program_id(1)

    @pl.when(jnp.logical_and(t == 0, h == 0))
    def _init():
        psum_ref[...] = jnp.zeros_like(psum_ref)

    z = z_ref[...]
    emb = emb_ref[0]

    zn = z / jnp.maximum(
        jnp.sqrt(jnp.sum(z * z, axis=-1, keepdims=True)), 1e-12)
    en = emb / jnp.maximum(
        jnp.sqrt(jnp.sum(emb * emb, axis=-1, keepdims=True)), 1e-12)

    raw = jax.lax.dot_general(
        zn, en, (((1,), (1,)), ((), ())),
        preferred_element_type=jnp.float32)

    m = jnp.max(raw, axis=-1, keepdims=True)
    niota = niota_ref[...]

    cand = jnp.where(raw >= m, niota, -3.0e38)
    widx = jnp.max(cand, axis=-1, keepdims=True)
    idx = (-widx[:, 0]).astype(jnp.int32)

    onehot = (cand == widx).astype(jnp.float32)
    zq_ref[...] = jax.lax.dot_general(
        onehot, emb, (((1,), (0,)), ((), ())),
        preferred_element_type=jnp.float32)

    k = scales_ref[h] * LOG2E
    e = jnp.exp2(raw * k)
    inv = 1.0 / jnp.sum(e, axis=-1, keepdims=True)
    colsum = jax.lax.dot_general(
        inv, e, (((0,), (0,)), ((), ())),
        preferred_element_type=jnp.float32)
    psum_ref[h, :] = psum_ref[h, :] + colsum[0]

    idx_ref[0, 0, :] = idx

    @pl.when(h == 0)
    def _comb0():
        comb_ref[0, 0, :] = idx

    @pl.when(h > 0)
    def _combh():
        comb_ref[0, 0, :] = comb_ref[0, 0, :] * CODES + idx

    @pl.when(jnp.logical_and(t == pl.num_programs(0) - 1, h == NH - 1))
    def _finish():
        p = psum_ref[...] * (1.0 / NTOK)
        ent = jnp.sum(p * jnp.log(p + 1e-10), axis=-1, keepdims=True)
        perp_ref[0, 0] = jnp.mean(jnp.exp(-ent))


@functools.partial(jax.jit, static_argnames=())
def kernel(z_e, embeddings, logit_scales):
    nt = NTOK // BT
    grid = (nt, NH)
    niota = -jnp.arange(CODES, dtype=jnp.float32).reshape(1, CODES)
    zq, idx, comb, perp = pl.pallas_call(
        _vq_kernel,
        grid=grid,
        in_specs=[
            pl.BlockSpec(memory_space=pltpu.SMEM),
            pl.BlockSpec((BT, HD), lambda t, h: (t, h)),
            pl.BlockSpec((1, CODES, HD), lambda t, h: (h, 0, 0)),
            pl.BlockSpec((1, CODES), lambda t, h: (0, 0)),
        ],
        out_specs=[
            pl.BlockSpec((BT, HD), lambda t, h: (t, h)),
            pl.BlockSpec((1, 1, BT), lambda t, h: (h, 0, t)),
            pl.BlockSpec((1, 1, BT), lambda t, h: (0, 0, t)),
            pl.BlockSpec((1, 1), lambda t, h: (0, 0),
                         memory_space=pltpu.SMEM),
        ],
        out_shape=[
            jax.ShapeDtypeStruct((NTOK, EMB), jnp.float32),
            jax.ShapeDtypeStruct((NH, 1, NTOK), jnp.int32),
            jax.ShapeDtypeStruct((1, 1, NTOK), jnp.int32),
            jax.ShapeDtypeStruct((1, 1), jnp.float32),
        ],
        scratch_shapes=[pltpu.VMEM((NH, CODES), jnp.float32)],
    )(logit_scales, z_e, embeddings, niota)

    temperature = jnp.asarray(1.0, dtype=jnp.float32)
    commitment_loss = jnp.asarray(0.0, dtype=jnp.float32)
    return (zq, comb[0, 0], perp[0, 0], temperature, commitment_loss)
